# baseline (speedup 1.0000x reference)
import jax
import jax.numpy as jnp
from jax import lax
from jax.experimental import pallas as pl
from jax.experimental.pallas import tpu as pltpu
from jax.experimental.pallas import tpu_sc as plsc

BATCH = 16384
EMB_DIM = 32
NUM_CORES = 2
NUM_SUBCORES = 16
LANES = 16
NUM_WORKERS = NUM_CORES * NUM_SUBCORES
BPW = BATCH // NUM_WORKERS
PANEL = 128
ROW = 128
CAP = BATCH + LANES
NBKT = 256
RING = 16
GROWS = BATCH + RING
PWAVE = 4


def _route_one(wid, tab_hbm, idx_hbm, g_hbm,
               idxsrt_v, rbuf_v, bbuf_v, srtb_v, upnl_v, ustt_v,
               hist_v, base_v, pan_v, ring_v, psem, wsem):
    iota = lax.iota(jnp.int32, LANES)
    ones = jnp.ones((LANES,), jnp.int32)
    zeros = jnp.zeros((LANES,), jnp.int32)
    dlo = iota
    dhi = iota + LANES
    rank0 = plsc.scan_count(zeros)[0][0]

    pltpu.sync_copy(idx_hbm, idxsrt_v.at[pl.ds(0, BATCH)])

    def p1(i, cursor):
        vec = idxsrt_v[pl.ds(i * LANES, LANES)]
        m = ((vec >> 7) & (NUM_WORKERS - 1)) == wid
        plsc.store_compressed(rbuf_v.at[pl.ds(cursor, LANES)], vec, mask=m)
        plsc.store_compressed(bbuf_v.at[pl.ds(cursor, LANES)],
                              i * LANES + iota, mask=m)
        return cursor + plsc.all_reduce_population_count(m)[0]

    cursor = lax.fori_loop(0, BATCH // LANES, p1, jnp.int32(0))
    ng = (cursor + LANES - 1) >> 4

    for j in range(NBKT // LANES):
        hist_v[pl.ds(j * LANES, LANES)] = zeros

    def p2(i, carry):
        vec = rbuf_v[pl.ds(i * LANES, LANES)]
        q = (vec >> 12) & (NBKT - 1)
        valid = (i * LANES + iota) < cursor
        plsc.addupdate_scatter(hist_v, [q], ones, mask=valid)
        return carry

    lax.fori_loop(0, ng, p2, 0)

    run = jnp.int32(0)
    for j in range(NBKT // LANES):
        vec = hist_v[pl.ds(j * LANES, LANES)]
        cs = plsc.cumsum(vec)
        base_v[pl.ds(j * LANES, LANES)] = (cs - vec) + run
        run = run + cs[LANES - 1]

    def p4(i, carry):
        vec_r = rbuf_v[pl.ds(i * LANES, LANES)]
        vec_b = bbuf_v[pl.ds(i * LANES, LANES)]
        valid = (i * LANES + iota) < cursor
        q = (vec_r >> 12) & (NBKT - 1)
        rk = plsc.scan_count(q, valid)[0] - rank0
        slot = plsc.load_gather(base_v, [q]) + rk
        plsc.store_scatter(idxsrt_v, [slot], vec_r, mask=valid)
        plsc.store_scatter(srtb_v, [slot], vec_b, mask=valid)
        plsc.addupdate_scatter(base_v, [q], ones, mask=valid)
        return carry

    lax.fori_loop(0, ng, p4, 0)

    idxsrt_v[pl.ds(cursor, LANES)] = zeros - 1

    def p5(i, carry):
        uc, prev_last = carry
        vec_p = idxsrt_v[pl.ds(i * LANES, LANES)] >> 7
        valid = (i * LANES + iota) < cursor
        cnts = plsc.scan_count(vec_p, valid)[0] - rank0
        newm = (cnts == 0) & (vec_p != prev_last) & valid
        plsc.store_compressed(upnl_v.at[pl.ds(uc, LANES)], vec_p, mask=newm)
        plsc.store_compressed(ustt_v.at[pl.ds(uc, LANES)],
                              i * LANES + iota, mask=newm)
        return (uc + plsc.all_reduce_population_count(newm)[0],
                vec_p[LANES - 1])

    uc, _ = lax.fori_loop(0, ng, p5, (jnp.int32(0), jnp.int32(-1)))
    ustt_v[pl.ds(uc, LANES)] = jnp.full((LANES,), cursor, jnp.int32)
    upnl_v[pl.ds(uc, LANES)] = zeros

    for t in range(RING):
        pltpu.async_copy(ring_v.at[t],
                         g_hbm.at[pl.ds((BATCH + t) * ROW, ROW)], wsem)

    def wave(w, carry):
        pans16 = upnl_v[pl.ds(w * PWAVE, LANES)]
        stts16 = ustt_v[pl.ds(w * PWAVE, LANES)]
        for k in range(PWAVE):
            off = pl.multiple_of(pans16[k] * PANEL, PANEL)
            pltpu.async_copy(tab_hbm.at[:, pl.ds(off, PANEL)],
                             pan_v.at[k], psem)
        for k in range(PWAVE):
            pltpu.make_async_copy(tab_hbm.at[:, pl.ds(0, PANEL)],
                                  pan_v.at[0], psem).wait()

        def entry(e, kcarry):
            kk = kcarry
            rv = idxsrt_v[pl.ds(e, LANES)]
            bv = srtb_v[pl.ds(e, LANES)]
            c = rv[0] & (PANEL - 1)
            b = bv[0]
            pltpu.make_async_copy(g_hbm.at[pl.ds(0, ROW)],
                                  ring_v.at[0], wsem).wait()
            sl = e & (RING - 1)
            kf = jnp.full((LANES,), kk, jnp.int32)
            cf = jnp.full((LANES,), c, jnp.int32)
            ring_v[sl, pl.ds(0, LANES)] = plsc.load_gather(pan_v, [kf, dlo, cf])
            ring_v[sl, pl.ds(LANES, LANES)] = plsc.load_gather(pan_v, [kf, dhi, cf])
            dst = pl.multiple_of(b * ROW, ROW)
            pltpu.async_copy(ring_v.at[sl], g_hbm.at[pl.ds(dst, ROW)], wsem)
            return kcarry

        for k in range(PWAVE):
            lax.fori_loop(stts16[k], stts16[k + 1], entry, k)
        return carry

    lax.fori_loop(0, (uc + PWAVE - 1) >> 2, wave, 0)

    for t in range(RING):
        pltpu.make_async_copy(g_hbm.at[pl.ds(0, ROW)],
                              ring_v.at[0], wsem).wait()


def _route_body(u_hbm, v_hbm, ut_hbm, vt_hbm, gu_hbm, gv_hbm,
                idxsrt_v, rbuf_v, bbuf_v, srtb_v, upnl_v, ustt_v,
                hist_v, base_v, pan_v, ring_v, psem, wsem):
    wid = lax.axis_index("s") * NUM_CORES + lax.axis_index("c")
    args = (idxsrt_v, rbuf_v, bbuf_v, srtb_v, upnl_v, ustt_v,
            hist_v, base_v, pan_v, ring_v, psem, wsem)
    _route_one(wid, ut_hbm, u_hbm, gu_hbm, *args)
    _route_one(wid, vt_hbm, v_hbm, gv_hbm, *args)


def _dot_body(gu_hbm, gv_hbm, out_hbm, gu_v, gv_v, out_v, sem):
    wid = lax.axis_index("s") * NUM_CORES + lax.axis_index("c")
    base = wid * BPW
    iota = lax.iota(jnp.int32, LANES)
    nch = BPW // 128

    for ch in range(nch):
        cb = (base + ch * 128) * ROW
        pltpu.sync_copy(gu_hbm.at[pl.ds(cb, 128 * ROW)], gu_v)
        pltpu.sync_copy(gv_hbm.at[pl.ds(cb, 128 * ROW)], gv_v)

        def group(g, carry):
            rows = (g * LANES + iota) * ROW
            acc = jnp.zeros((LANES,), jnp.float32)
            for d in range(EMB_DIM):
                idx = rows + d
                acc = acc + plsc.load_gather(gu_v, [idx]) * plsc.load_gather(gv_v, [idx])
            out_v[pl.ds(ch * 128 + g * LANES, LANES)] = (
                4.0 / (1.0 + jnp.exp(-acc)) + 1.0)
            return carry

        lax.fori_loop(0, 128 // LANES, group, 0)

    pltpu.sync_copy(out_v, out_hbm.at[pl.ds(base, BPW)])


def kernel(u, v, u_table, v_table):
    mesh = plsc.VectorSubcoreMesh(core_axis_name="c", subcore_axis_name="s")
    cparams = pltpu.CompilerParams(
        needs_layout_passes=False, use_tc_tiling_on_sc=True)

    k1 = pl.kernel(
        _route_body,
        mesh=mesh,
        compiler_params=cparams,
        out_type=(jax.ShapeDtypeStruct((GROWS * ROW,), jnp.float32),
                  jax.ShapeDtypeStruct((GROWS * ROW,), jnp.float32)),
        scratch_types=[
            pltpu.VMEM((CAP,), jnp.int32),
            pltpu.VMEM((CAP,), jnp.int32),
            pltpu.VMEM((CAP,), jnp.int32),
            pltpu.VMEM((CAP,), jnp.int32),
            pltpu.VMEM((CAP,), jnp.int32),
            pltpu.VMEM((CAP,), jnp.int32),
            pltpu.VMEM((NBKT,), jnp.int32),
            pltpu.VMEM((NBKT,), jnp.int32),
            pltpu.VMEM((PWAVE, EMB_DIM, PANEL), jnp.float32),
            pltpu.VMEM((RING, ROW), jnp.float32),
            pltpu.SemaphoreType.DMA,
            pltpu.SemaphoreType.DMA,
        ],
    )
    gu, gv = k1(u, v, u_table.T, v_table.T)

    k2 = pl.kernel(
        _dot_body,
        mesh=mesh,
        compiler_params=cparams,
        out_type=jax.ShapeDtypeStruct((BATCH,), jnp.float32),
        scratch_types=[
            pltpu.VMEM((128 * ROW,), jnp.float32),
            pltpu.VMEM((128 * ROW,), jnp.float32),
            pltpu.VMEM((BPW,), jnp.float32),
            pltpu.SemaphoreType.DMA,
        ],
    )
    return k2(gu, gv)


# speedup vs baseline: 1.2217x; 1.2217x over previous
import jax
import jax.numpy as jnp
from jax import lax
from jax.experimental import pallas as pl
from jax.experimental.pallas import tpu as pltpu
from jax.experimental.pallas import tpu_sc as plsc

BATCH = 16384
EMB_DIM = 32
NUM_CORES = 2
NUM_SUBCORES = 16
LANES = 16
NUM_WORKERS = NUM_CORES * NUM_SUBCORES
BPW = BATCH // NUM_WORKERS
PANEL = 128
ROW = 128
CAP = BATCH + LANES
NBKT = 256
RING = 16
GROWS = BATCH + RING
PWAVE = 2


def _route_one(wid, tab_hbm, idx_hbm, g_hbm,
               idxsrt_v, rbuf_v, bbuf_v, srtb_v, upnl_v, ustt_v,
               hist_v, base_v, pan_v, ring_v, psem, psem2, wsem):
    iota = lax.iota(jnp.int32, LANES)
    ones = jnp.ones((LANES,), jnp.int32)
    zeros = jnp.zeros((LANES,), jnp.int32)
    dlo = iota
    dhi = iota + LANES
    rank0 = plsc.scan_count(zeros)[0][0]

    pltpu.sync_copy(idx_hbm, idxsrt_v.at[pl.ds(0, BATCH)])

    def p1(i, cursor):
        vec = idxsrt_v[pl.ds(i * LANES, LANES)]
        m = ((vec >> 7) & (NUM_WORKERS - 1)) == wid
        plsc.store_compressed(rbuf_v.at[pl.ds(cursor, LANES)], vec, mask=m)
        plsc.store_compressed(bbuf_v.at[pl.ds(cursor, LANES)],
                              i * LANES + iota, mask=m)
        return cursor + plsc.all_reduce_population_count(m)[0]

    cursor = lax.fori_loop(0, BATCH // LANES, p1, jnp.int32(0))
    ng = (cursor + LANES - 1) >> 4

    for j in range(NBKT // LANES):
        hist_v[pl.ds(j * LANES, LANES)] = zeros

    def p2(i, carry):
        vec = rbuf_v[pl.ds(i * LANES, LANES)]
        q = (vec >> 12) & (NBKT - 1)
        valid = (i * LANES + iota) < cursor
        plsc.addupdate_scatter(hist_v, [q], ones, mask=valid)
        return carry

    lax.fori_loop(0, ng, p2, 0)

    run = jnp.int32(0)
    for j in range(NBKT // LANES):
        vec = hist_v[pl.ds(j * LANES, LANES)]
        cs = plsc.cumsum(vec)
        base_v[pl.ds(j * LANES, LANES)] = (cs - vec) + run
        run = run + cs[LANES - 1]

    def p4(i, carry):
        vec_r = rbuf_v[pl.ds(i * LANES, LANES)]
        vec_b = bbuf_v[pl.ds(i * LANES, LANES)]
        valid = (i * LANES + iota) < cursor
        q = (vec_r >> 12) & (NBKT - 1)
        rk = plsc.scan_count(q, valid)[0] - rank0
        slot = plsc.load_gather(base_v, [q]) + rk
        plsc.store_scatter(idxsrt_v, [slot], vec_r, mask=valid)
        plsc.store_scatter(srtb_v, [slot], vec_b, mask=valid)
        plsc.addupdate_scatter(base_v, [q], ones, mask=valid)
        return carry

    lax.fori_loop(0, ng, p4, 0)

    idxsrt_v[pl.ds(cursor, LANES)] = zeros - 1

    def p5(i, carry):
        uc, prev_last = carry
        vec_p = idxsrt_v[pl.ds(i * LANES, LANES)] >> 7
        valid = (i * LANES + iota) < cursor
        cnts = plsc.scan_count(vec_p, valid)[0] - rank0
        newm = (cnts == 0) & (vec_p != prev_last) & valid
        plsc.store_compressed(upnl_v.at[pl.ds(uc, LANES)], vec_p, mask=newm)
        plsc.store_compressed(ustt_v.at[pl.ds(uc, LANES)],
                              i * LANES + iota, mask=newm)
        return (uc + plsc.all_reduce_population_count(newm)[0],
                vec_p[LANES - 1])

    uc, _ = lax.fori_loop(0, ng, p5, (jnp.int32(0), jnp.int32(-1)))
    ustt_v[pl.ds(uc, LANES)] = jnp.full((LANES,), cursor, jnp.int32)
    upnl_v[pl.ds(uc, LANES)] = zeros

    for t in range(RING):
        pltpu.async_copy(ring_v.at[t],
                         g_hbm.at[pl.ds((BATCH + t) * ROW, ROW)], wsem)

    nw = (uc + PWAVE - 1) >> 1

    def fire(w, par, sem):
        @pl.when(w < nw)
        def _():
            pans16 = upnl_v[pl.ds(w * PWAVE, LANES)]
            for k in range(PWAVE):
                off = pl.multiple_of(pans16[k] * PANEL, PANEL)
                pltpu.async_copy(tab_hbm.at[:, pl.ds(off, PANEL)],
                                 pan_v.at[par * PWAVE + k], sem)

    def drain_extract(w, par, sem):
        @pl.when(w < nw)
        def _():
            for k in range(PWAVE):
                pltpu.make_async_copy(tab_hbm.at[:, pl.ds(0, PANEL)],
                                      pan_v.at[0], sem).wait()
            stts16 = ustt_v[pl.ds(w * PWAVE, LANES)]

            def entry(e, kcarry):
                rv = idxsrt_v[pl.ds(e, LANES)]
                bv = srtb_v[pl.ds(e, LANES)]
                c = rv[0] & (PANEL - 1)
                b = bv[0]
                pltpu.make_async_copy(g_hbm.at[pl.ds(0, ROW)],
                                      ring_v.at[0], wsem).wait()
                sl = e & (RING - 1)
                kf = jnp.full((LANES,), par * PWAVE + kcarry, jnp.int32)
                cf = jnp.full((LANES,), c, jnp.int32)
                ring_v[sl, pl.ds(0, LANES)] = plsc.load_gather(
                    pan_v, [kf, dlo, cf])
                ring_v[sl, pl.ds(LANES, LANES)] = plsc.load_gather(
                    pan_v, [kf, dhi, cf])
                dst = pl.multiple_of(b * ROW, ROW)
                pltpu.async_copy(ring_v.at[sl],
                                 g_hbm.at[pl.ds(dst, ROW)], wsem)
                return kcarry

            for k in range(PWAVE):
                lax.fori_loop(stts16[k], stts16[k + 1], entry, k)

    fire(jnp.int32(0), 0, psem)

    def pair(p, carry):
        w0 = p * 2
        fire(w0 + 1, 1, psem2)
        drain_extract(w0, 0, psem)
        fire(w0 + 2, 0, psem)
        drain_extract(w0 + 1, 1, psem2)
        return carry

    lax.fori_loop(0, (nw + 1) >> 1, pair, 0)

    for t in range(RING):
        pltpu.make_async_copy(g_hbm.at[pl.ds(0, ROW)],
                              ring_v.at[0], wsem).wait()


def _route_body(u_hbm, v_hbm, ut_hbm, vt_hbm, gu_hbm, gv_hbm,
                idxsrt_v, rbuf_v, bbuf_v, srtb_v, upnl_v, ustt_v,
                hist_v, base_v, pan_v, ring_v, psem, psem2, wsem):
    wid = lax.axis_index("s") * NUM_CORES + lax.axis_index("c")
    args = (idxsrt_v, rbuf_v, bbuf_v, srtb_v, upnl_v, ustt_v,
            hist_v, base_v, pan_v, ring_v, psem, psem2, wsem)
    _route_one(wid, ut_hbm, u_hbm, gu_hbm, *args)
    _route_one(wid, vt_hbm, v_hbm, gv_hbm, *args)


def _dot_body(gu_hbm, gv_hbm, out_hbm, gu_v, gv_v, out_v, sem):
    wid = lax.axis_index("s") * NUM_CORES + lax.axis_index("c")
    base = wid * BPW
    iota = lax.iota(jnp.int32, LANES)
    nch = BPW // 128

    for ch in range(nch):
        cb = (base + ch * 128) * ROW
        pltpu.sync_copy(gu_hbm.at[pl.ds(cb, 128 * ROW)], gu_v)
        pltpu.sync_copy(gv_hbm.at[pl.ds(cb, 128 * ROW)], gv_v)

        def group(g, carry):
            rows = (g * LANES + iota) * ROW
            acc = jnp.zeros((LANES,), jnp.float32)
            for d in range(EMB_DIM):
                idx = rows + d
                acc = acc + plsc.load_gather(gu_v, [idx]) * plsc.load_gather(gv_v, [idx])
            out_v[pl.ds(ch * 128 + g * LANES, LANES)] = (
                4.0 / (1.0 + jnp.exp(-acc)) + 1.0)
            return carry

        lax.fori_loop(0, 128 // LANES, group, 0)

    pltpu.sync_copy(out_v, out_hbm.at[pl.ds(base, BPW)])


def kernel(u, v, u_table, v_table):
    mesh = plsc.VectorSubcoreMesh(core_axis_name="c", subcore_axis_name="s")
    cparams = pltpu.CompilerParams(
        needs_layout_passes=False, use_tc_tiling_on_sc=True)

    k1 = pl.kernel(
        _route_body,
        mesh=mesh,
        compiler_params=cparams,
        out_type=(jax.ShapeDtypeStruct((GROWS * ROW,), jnp.float32),
                  jax.ShapeDtypeStruct((GROWS * ROW,), jnp.float32)),
        scratch_types=[
            pltpu.VMEM((CAP,), jnp.int32),
            pltpu.VMEM((CAP,), jnp.int32),
            pltpu.VMEM((CAP,), jnp.int32),
            pltpu.VMEM((CAP,), jnp.int32),
            pltpu.VMEM((CAP,), jnp.int32),
            pltpu.VMEM((CAP,), jnp.int32),
            pltpu.VMEM((NBKT,), jnp.int32),
            pltpu.VMEM((NBKT,), jnp.int32),
            pltpu.VMEM((2 * PWAVE, EMB_DIM, PANEL), jnp.float32),
            pltpu.VMEM((RING, ROW), jnp.float32),
            pltpu.SemaphoreType.DMA,
            pltpu.SemaphoreType.DMA,
            pltpu.SemaphoreType.DMA,
        ],
    )
    gu, gv = k1(u, v, u_table.T, v_table.T)

    k2 = pl.kernel(
        _dot_body,
        mesh=mesh,
        compiler_params=cparams,
        out_type=jax.ShapeDtypeStruct((BATCH,), jnp.float32),
        scratch_types=[
            pltpu.VMEM((128 * ROW,), jnp.float32),
            pltpu.VMEM((128 * ROW,), jnp.float32),
            pltpu.VMEM((BPW,), jnp.float32),
            pltpu.SemaphoreType.DMA,
        ],
    )
    return k2(gu, gv)


# speedup vs baseline: 1.4160x; 1.1590x over previous
import jax
import jax.numpy as jnp
from jax import lax
from jax.experimental import pallas as pl
from jax.experimental.pallas import tpu as pltpu
from jax.experimental.pallas import tpu_sc as plsc

BATCH = 16384
EMB_DIM = 32
NUM_CORES = 2
NUM_SUBCORES = 16
LANES = 16
NUM_WORKERS = NUM_CORES * NUM_SUBCORES
BPW = BATCH // NUM_WORKERS
PANEL = 128
ROW = 128
CAP = BATCH + LANES
NBKT = 256
RING = 16
GROWS = BATCH + RING
PWAVE = 4


def _route_one(wid, tab_hbm, idx_hbm, g_hbm,
               idxsrt_v, rbuf_v, bbuf_v, srtb_v, upk_v,
               hist_v, base_v, pan_v, ring_v, psem, psem2, wsem):
    iota = lax.iota(jnp.int32, LANES)
    ones = jnp.ones((LANES,), jnp.int32)
    zeros = jnp.zeros((LANES,), jnp.int32)
    dlo = iota
    dhi = iota + LANES
    rank0 = plsc.scan_count(zeros)[0][0]

    pltpu.sync_copy(idx_hbm, idxsrt_v.at[pl.ds(0, BATCH)])

    def p1(i, cursor):
        vec = idxsrt_v[pl.ds(i * LANES, LANES)]
        m = ((vec >> 7) & (NUM_WORKERS - 1)) == wid
        plsc.store_compressed(rbuf_v.at[pl.ds(cursor, LANES)], vec, mask=m)
        plsc.store_compressed(bbuf_v.at[pl.ds(cursor, LANES)],
                              i * LANES + iota, mask=m)
        return cursor + plsc.all_reduce_population_count(m)[0]

    cursor = lax.fori_loop(0, BATCH // LANES, p1, jnp.int32(0))
    ng = (cursor + LANES - 1) >> 4

    for j in range(NBKT // LANES):
        hist_v[pl.ds(j * LANES, LANES)] = zeros

    def p2(i, carry):
        vec = rbuf_v[pl.ds(i * LANES, LANES)]
        q = (vec >> 12) & (NBKT - 1)
        valid = (i * LANES + iota) < cursor
        plsc.addupdate_scatter(hist_v, [q], ones, mask=valid)
        return carry

    lax.fori_loop(0, ng, p2, 0)

    run = jnp.int32(0)
    for j in range(NBKT // LANES):
        vec = hist_v[pl.ds(j * LANES, LANES)]
        cs = plsc.cumsum(vec)
        base_v[pl.ds(j * LANES, LANES)] = (cs - vec) + run
        run = run + cs[LANES - 1]

    def p4(i, carry):
        vec_r = rbuf_v[pl.ds(i * LANES, LANES)]
        vec_b = bbuf_v[pl.ds(i * LANES, LANES)]
        valid = (i * LANES + iota) < cursor
        q = (vec_r >> 12) & (NBKT - 1)
        rk = plsc.scan_count(q, valid)[0] - rank0
        slot = plsc.load_gather(base_v, [q]) + rk
        plsc.store_scatter(idxsrt_v, [slot], vec_r, mask=valid)
        plsc.store_scatter(srtb_v, [slot], vec_b, mask=valid)
        plsc.addupdate_scatter(base_v, [q], ones, mask=valid)
        return carry

    lax.fori_loop(0, ng, p4, 0)

    idxsrt_v[pl.ds(cursor, LANES)] = zeros - 1

    def p5(i, carry):
        uc, prev_last = carry
        vec_p = idxsrt_v[pl.ds(i * LANES, LANES)] >> 7
        valid = (i * LANES + iota) < cursor
        cnts = plsc.scan_count(vec_p, valid)[0] - rank0
        newm = (cnts == 0) & (vec_p != prev_last) & valid
        pk = ((i * LANES + iota) << 13) | vec_p
        plsc.store_compressed(upk_v.at[pl.ds(uc, LANES)], pk, mask=newm)
        return (uc + plsc.all_reduce_population_count(newm)[0],
                vec_p[LANES - 1])

    uc, _ = lax.fori_loop(0, ng, p5, (jnp.int32(0), jnp.int32(-1)))
    upk_v[pl.ds(uc, LANES)] = jnp.full((LANES,), cursor << 13, jnp.int32)

    for t in range(RING):
        pltpu.async_copy(ring_v.at[t],
                         g_hbm.at[pl.ds((BATCH + t) * ROW, ROW)], wsem)

    nw = (uc + PWAVE - 1) >> 2

    def fire(w, par, sem):
        @pl.when(w < nw)
        def _():
            pans16 = upk_v[pl.ds(w * PWAVE, LANES)] & (8192 - 1)
            for k in range(PWAVE):
                off = pl.multiple_of(pans16[k] * PANEL, PANEL)
                pltpu.async_copy(tab_hbm.at[:, pl.ds(off, PANEL)],
                                 pan_v.at[par * PWAVE + k], sem)

    def drain_extract(w, par, sem):
        @pl.when(w < nw)
        def _():
            for k in range(PWAVE):
                pltpu.make_async_copy(tab_hbm.at[:, pl.ds(0, PANEL)],
                                      pan_v.at[0], sem).wait()
            stts16 = lax.shift_right_logical(upk_v[pl.ds(w * PWAVE, LANES)], 13)

            def entry(e, kcarry):
                rv = idxsrt_v[pl.ds(e, LANES)]
                bv = srtb_v[pl.ds(e, LANES)]
                c = rv[0] & (PANEL - 1)
                b = bv[0]
                pltpu.make_async_copy(g_hbm.at[pl.ds(0, ROW)],
                                      ring_v.at[0], wsem).wait()
                sl = e & (RING - 1)
                kf = jnp.full((LANES,), par * PWAVE + kcarry, jnp.int32)
                cf = jnp.full((LANES,), c, jnp.int32)
                ring_v[sl, pl.ds(0, LANES)] = plsc.load_gather(
                    pan_v, [kf, dlo, cf])
                ring_v[sl, pl.ds(LANES, LANES)] = plsc.load_gather(
                    pan_v, [kf, dhi, cf])
                dst = pl.multiple_of(b * ROW, ROW)
                pltpu.async_copy(ring_v.at[sl],
                                 g_hbm.at[pl.ds(dst, ROW)], wsem)
                return kcarry

            for k in range(PWAVE):
                lax.fori_loop(stts16[k], stts16[k + 1], entry, k)

    fire(jnp.int32(0), 0, psem)

    def pair(p, carry):
        w0 = p * 2
        fire(w0 + 1, 1, psem2)
        drain_extract(w0, 0, psem)
        fire(w0 + 2, 0, psem)
        drain_extract(w0 + 1, 1, psem2)
        return carry

    lax.fori_loop(0, (nw + 1) >> 1, pair, 0)

    for t in range(RING):
        pltpu.make_async_copy(g_hbm.at[pl.ds(0, ROW)],
                              ring_v.at[0], wsem).wait()


def _route_body(u_hbm, v_hbm, ut_hbm, vt_hbm, gu_hbm, gv_hbm,
                idxsrt_v, rbuf_v, bbuf_v, srtb_v, upk_v,
                hist_v, base_v, pan_v, ring_v, psem, psem2, wsem):
    wid = lax.axis_index("s") * NUM_CORES + lax.axis_index("c")
    args = (idxsrt_v, rbuf_v, bbuf_v, srtb_v, upk_v,
            hist_v, base_v, pan_v, ring_v, psem, psem2, wsem)
    _route_one(wid, ut_hbm, u_hbm, gu_hbm, *args)
    _route_one(wid, vt_hbm, v_hbm, gv_hbm, *args)


def _dot_body(gu_hbm, gv_hbm, out_hbm, gu_v, gv_v, out_v, sem):
    wid = lax.axis_index("s") * NUM_CORES + lax.axis_index("c")
    base = wid * BPW
    iota = lax.iota(jnp.int32, LANES)
    nch = BPW // 128

    for ch in range(nch):
        cb = (base + ch * 128) * ROW
        pltpu.sync_copy(gu_hbm.at[pl.ds(cb, 128 * ROW)], gu_v)
        pltpu.sync_copy(gv_hbm.at[pl.ds(cb, 128 * ROW)], gv_v)

        def group(g, carry):
            rows = (g * LANES + iota) * ROW
            acc = jnp.zeros((LANES,), jnp.float32)
            for d in range(EMB_DIM):
                idx = rows + d
                acc = acc + plsc.load_gather(gu_v, [idx]) * plsc.load_gather(gv_v, [idx])
            out_v[pl.ds(ch * 128 + g * LANES, LANES)] = (
                4.0 / (1.0 + jnp.exp(-acc)) + 1.0)
            return carry

        lax.fori_loop(0, 128 // LANES, group, 0)

    pltpu.sync_copy(out_v, out_hbm.at[pl.ds(base, BPW)])


def kernel(u, v, u_table, v_table):
    mesh = plsc.VectorSubcoreMesh(core_axis_name="c", subcore_axis_name="s")
    cparams = pltpu.CompilerParams(
        needs_layout_passes=False, use_tc_tiling_on_sc=True)

    k1 = pl.kernel(
        _route_body,
        mesh=mesh,
        compiler_params=cparams,
        out_type=(jax.ShapeDtypeStruct((GROWS * ROW,), jnp.float32),
                  jax.ShapeDtypeStruct((GROWS * ROW,), jnp.float32)),
        scratch_types=[
            pltpu.VMEM((CAP,), jnp.int32),
            pltpu.VMEM((CAP,), jnp.int32),
            pltpu.VMEM((CAP,), jnp.int32),
            pltpu.VMEM((CAP,), jnp.int32),
            pltpu.VMEM((CAP,), jnp.int32),
            pltpu.VMEM((NBKT,), jnp.int32),
            pltpu.VMEM((NBKT,), jnp.int32),
            pltpu.VMEM((2 * PWAVE, EMB_DIM, PANEL), jnp.float32),
            pltpu.VMEM((RING, ROW), jnp.float32),
            pltpu.SemaphoreType.DMA,
            pltpu.SemaphoreType.DMA,
            pltpu.SemaphoreType.DMA,
        ],
    )
    gu, gv = k1(u, v, u_table.T, v_table.T)

    k2 = pl.kernel(
        _dot_body,
        mesh=mesh,
        compiler_params=cparams,
        out_type=jax.ShapeDtypeStruct((BATCH,), jnp.float32),
        scratch_types=[
            pltpu.VMEM((128 * ROW,), jnp.float32),
            pltpu.VMEM((128 * ROW,), jnp.float32),
            pltpu.VMEM((BPW,), jnp.float32),
            pltpu.SemaphoreType.DMA,
        ],
    )
    return k2(gu, gv)


# speedup vs baseline: 1.4644x; 1.0342x over previous
import jax
import jax.numpy as jnp
from jax import lax
from jax.experimental import pallas as pl
from jax.experimental.pallas import tpu as pltpu
from jax.experimental.pallas import tpu_sc as plsc

BATCH = 16384
EMB_DIM = 32
NUM_CORES = 2
NUM_SUBCORES = 16
LANES = 16
NUM_WORKERS = NUM_CORES * NUM_SUBCORES
BPW = BATCH // NUM_WORKERS
PANEL = 128
ROW = 32
CAP = BATCH + LANES
NBKT = 256
RING = 16
GROWS = BATCH + RING
PWAVE = 4


def _route_one(wid, tab_hbm, idx_hbm, g_hbm,
               idxsrt_v, rbuf_v, bbuf_v, srtb_v, upk_v,
               hist_v, base_v, pan_v, ring_v, psem, psem2, wsem):
    iota = lax.iota(jnp.int32, LANES)
    ones = jnp.ones((LANES,), jnp.int32)
    zeros = jnp.zeros((LANES,), jnp.int32)
    dlo = iota
    dhi = iota + LANES
    rank0 = plsc.scan_count(zeros)[0][0]

    pltpu.sync_copy(idx_hbm, idxsrt_v.at[pl.ds(0, BATCH)])

    def p1(i, cursor):
        vec = idxsrt_v[pl.ds(i * LANES, LANES)]
        m = ((vec >> 7) & (NUM_WORKERS - 1)) == wid
        plsc.store_compressed(rbuf_v.at[pl.ds(cursor, LANES)], vec, mask=m)
        plsc.store_compressed(bbuf_v.at[pl.ds(cursor, LANES)],
                              i * LANES + iota, mask=m)
        return cursor + plsc.all_reduce_population_count(m)[0]

    cursor = lax.fori_loop(0, BATCH // LANES, p1, jnp.int32(0))
    ng = (cursor + LANES - 1) >> 4

    for j in range(NBKT // LANES):
        hist_v[pl.ds(j * LANES, LANES)] = zeros

    def p2(i, carry):
        vec = rbuf_v[pl.ds(i * LANES, LANES)]
        q = (vec >> 12) & (NBKT - 1)
        valid = (i * LANES + iota) < cursor
        plsc.addupdate_scatter(hist_v, [q], ones, mask=valid)
        return carry

    lax.fori_loop(0, ng, p2, 0)

    run = jnp.int32(0)
    for j in range(NBKT // LANES):
        vec = hist_v[pl.ds(j * LANES, LANES)]
        cs = plsc.cumsum(vec)
        base_v[pl.ds(j * LANES, LANES)] = (cs - vec) + run
        run = run + cs[LANES - 1]

    def p4(i, carry):
        vec_r = rbuf_v[pl.ds(i * LANES, LANES)]
        vec_b = bbuf_v[pl.ds(i * LANES, LANES)]
        valid = (i * LANES + iota) < cursor
        q = (vec_r >> 12) & (NBKT - 1)
        rk = plsc.scan_count(q, valid)[0] - rank0
        slot = plsc.load_gather(base_v, [q]) + rk
        plsc.store_scatter(idxsrt_v, [slot], vec_r, mask=valid)
        plsc.store_scatter(srtb_v, [slot], vec_b, mask=valid)
        plsc.addupdate_scatter(base_v, [q], ones, mask=valid)
        return carry

    lax.fori_loop(0, ng, p4, 0)

    idxsrt_v[pl.ds(cursor, LANES)] = zeros - 1

    def p5(i, carry):
        uc, prev_last = carry
        vec_p = idxsrt_v[pl.ds(i * LANES, LANES)] >> 7
        valid = (i * LANES + iota) < cursor
        cnts = plsc.scan_count(vec_p, valid)[0] - rank0
        newm = (cnts == 0) & (vec_p != prev_last) & valid
        pk = ((i * LANES + iota) << 13) | vec_p
        plsc.store_compressed(upk_v.at[pl.ds(uc, LANES)], pk, mask=newm)
        return (uc + plsc.all_reduce_population_count(newm)[0],
                vec_p[LANES - 1])

    uc, _ = lax.fori_loop(0, ng, p5, (jnp.int32(0), jnp.int32(-1)))
    upk_v[pl.ds(uc, LANES)] = jnp.full((LANES,), cursor << 13, jnp.int32)

    for t in range(RING):
        pltpu.async_copy(ring_v.at[t],
                         g_hbm.at[pl.ds((BATCH + t) * ROW, ROW)], wsem)

    nw = (uc + PWAVE - 1) >> 2

    def fire(w, par, sem):
        @pl.when(w < nw)
        def _():
            pans16 = upk_v[pl.ds(w * PWAVE, LANES)] & (8192 - 1)
            for k in range(PWAVE):
                off = pl.multiple_of(pans16[k] * PANEL, PANEL)
                pltpu.async_copy(tab_hbm.at[:, pl.ds(off, PANEL)],
                                 pan_v.at[par * PWAVE + k], sem)

    def drain_extract(w, par, sem):
        @pl.when(w < nw)
        def _():
            for k in range(PWAVE):
                pltpu.make_async_copy(tab_hbm.at[:, pl.ds(0, PANEL)],
                                      pan_v.at[0], sem).wait()
            stts16 = lax.shift_right_logical(upk_v[pl.ds(w * PWAVE, LANES)], 13)

            def entry(e, kcarry):
                rv = idxsrt_v[pl.ds(e, LANES)]
                bv = srtb_v[pl.ds(e, LANES)]
                c = rv[0] & (PANEL - 1)
                b = bv[0]
                pltpu.make_async_copy(g_hbm.at[pl.ds(0, ROW)],
                                      ring_v.at[0], wsem).wait()
                sl = e & (RING - 1)
                kf = jnp.full((LANES,), par * PWAVE + kcarry, jnp.int32)
                cf = jnp.full((LANES,), c, jnp.int32)
                ring_v[sl, pl.ds(0, LANES)] = plsc.load_gather(
                    pan_v, [kf, dlo, cf])
                ring_v[sl, pl.ds(LANES, LANES)] = plsc.load_gather(
                    pan_v, [kf, dhi, cf])
                dst = pl.multiple_of(b * ROW, ROW)
                pltpu.async_copy(ring_v.at[sl],
                                 g_hbm.at[pl.ds(dst, ROW)], wsem)
                return kcarry

            for k in range(PWAVE):
                lax.fori_loop(stts16[k], stts16[k + 1], entry, k)

    fire(jnp.int32(0), 0, psem)

    def pair(p, carry):
        w0 = p * 2
        fire(w0 + 1, 1, psem2)
        drain_extract(w0, 0, psem)
        fire(w0 + 2, 0, psem)
        drain_extract(w0 + 1, 1, psem2)
        return carry

    lax.fori_loop(0, (nw + 1) >> 1, pair, 0)

    for t in range(RING):
        pltpu.make_async_copy(g_hbm.at[pl.ds(0, ROW)],
                              ring_v.at[0], wsem).wait()


def _route_body(u_hbm, v_hbm, ut_hbm, vt_hbm, gu_hbm, gv_hbm,
                idxsrt_v, rbuf_v, bbuf_v, srtb_v, upk_v,
                hist_v, base_v, pan_v, ring_v, psem, psem2, wsem):
    wid = lax.axis_index("s") * NUM_CORES + lax.axis_index("c")
    args = (idxsrt_v, rbuf_v, bbuf_v, srtb_v, upk_v,
            hist_v, base_v, pan_v, ring_v, psem, psem2, wsem)
    _route_one(wid, ut_hbm, u_hbm, gu_hbm, *args)
    _route_one(wid, vt_hbm, v_hbm, gv_hbm, *args)


def _dot_body(gu_hbm, gv_hbm, out_hbm, gu_v, gv_v, out_v, sem):
    wid = lax.axis_index("s") * NUM_CORES + lax.axis_index("c")
    base = wid * BPW
    iota = lax.iota(jnp.int32, LANES)
    nch = BPW // 128

    for ch in range(nch):
        cb = (base + ch * 128) * ROW
        pltpu.sync_copy(gu_hbm.at[pl.ds(cb, 128 * ROW)], gu_v)
        pltpu.sync_copy(gv_hbm.at[pl.ds(cb, 128 * ROW)], gv_v)

        def group(g, carry):
            rows = (g * LANES + iota) * ROW
            acc = jnp.zeros((LANES,), jnp.float32)
            for d in range(EMB_DIM):
                idx = rows + d
                acc = acc + plsc.load_gather(gu_v, [idx]) * plsc.load_gather(gv_v, [idx])
            out_v[pl.ds(ch * 128 + g * LANES, LANES)] = (
                4.0 / (1.0 + jnp.exp(-acc)) + 1.0)
            return carry

        lax.fori_loop(0, 128 // LANES, group, 0)

    pltpu.sync_copy(out_v, out_hbm.at[pl.ds(base, BPW)])


def kernel(u, v, u_table, v_table):
    mesh = plsc.VectorSubcoreMesh(core_axis_name="c", subcore_axis_name="s")
    cparams = pltpu.CompilerParams(
        needs_layout_passes=False, use_tc_tiling_on_sc=True)

    k1 = pl.kernel(
        _route_body,
        mesh=mesh,
        compiler_params=cparams,
        out_type=(jax.ShapeDtypeStruct((GROWS * ROW,), jnp.float32),
                  jax.ShapeDtypeStruct((GROWS * ROW,), jnp.float32)),
        scratch_types=[
            pltpu.VMEM((CAP,), jnp.int32),
            pltpu.VMEM((CAP,), jnp.int32),
            pltpu.VMEM((CAP,), jnp.int32),
            pltpu.VMEM((CAP,), jnp.int32),
            pltpu.VMEM((CAP,), jnp.int32),
            pltpu.VMEM((NBKT,), jnp.int32),
            pltpu.VMEM((NBKT,), jnp.int32),
            pltpu.VMEM((2 * PWAVE, EMB_DIM, PANEL), jnp.float32),
            pltpu.VMEM((RING, ROW), jnp.float32),
            pltpu.SemaphoreType.DMA,
            pltpu.SemaphoreType.DMA,
            pltpu.SemaphoreType.DMA,
        ],
    )
    gu, gv = k1(u, v, u_table.T, v_table.T)

    k2 = pl.kernel(
        _dot_body,
        mesh=mesh,
        compiler_params=cparams,
        out_type=jax.ShapeDtypeStruct((BATCH,), jnp.float32),
        scratch_types=[
            pltpu.VMEM((128 * ROW,), jnp.float32),
            pltpu.VMEM((128 * ROW,), jnp.float32),
            pltpu.VMEM((BPW,), jnp.float32),
            pltpu.SemaphoreType.DMA,
        ],
    )
    return k2(gu, gv)
